# Initial kernel scaffold; baseline (speedup 1.0000x reference)
import jax
import jax.numpy as jnp
from jax.experimental import pallas as pl


def kernel(x_o, x_a, W1, b1, a1, W2, b2, a2, Wm0, bm0, Wm1, bm1, Wd1, bd1, Wd2, bd2, Wb, bb, edge_index, idx):
    raise NotImplementedError("write your pallas kernel here")


# speedup vs baseline: 21.2138x; 21.2138x over previous
import functools

import jax
import jax.numpy as jnp
from jax import lax
from jax.experimental import pallas as pl
from jax.experimental.pallas import tpu as pltpu
from jax.experimental.pallas import tpu_sc as plsc

N = 10000
F_IN = 128
H1 = 64
H2 = 32
B = 4096

NC = 2
NS = 16
NW = NC * NS

CH = 128
NCHUNKS = 2560
EPAD = NCHUNKS * CH
E = 320000
CPT_DEG = NCHUNKS // NW
CPT_AGG = NCHUNKS // NS

ROWS_PT = 632
NPAD = NS * ROWS_PT
DUMMY = N
DEGW = 16

_mesh = plsc.VectorSubcoreMesh(
    core_axis_name="c", subcore_axis_name="s", num_cores=NC, num_subcores=NS)


@functools.partial(
    pl.kernel,
    out_type=jax.ShapeDtypeStruct((NC, NPAD, DEGW), jnp.float32),
    mesh=_mesh,
    compiler_params=pltpu.CompilerParams(use_tc_tiling_on_sc=False),
    scratch_types=[
        pltpu.VMEM((CPT_DEG, CH), jnp.int32),
        pltpu.VMEM((CH, DEGW), jnp.float32),
        pltpu.VMEM((ROWS_PT, DEGW), jnp.float32),
        pltpu.VMEM_SHARED((NPAD, DEGW), jnp.float32),
    ],
)
def _deg_kernel(dst_hbm, out_hbm, dst_v, ones_v, zero_v, acc_sh):
    c = lax.axis_index("c")
    s = lax.axis_index("s")
    wid = s * NC + c
    base = s * ROWS_PT

    def fill_ones(i, _):
        ones_v[i] = jnp.full((16,), 1.0, jnp.float32)
        return 0

    lax.fori_loop(0, CH, fill_ones, 0)

    def fill_zeros(i, _):
        zero_v[i] = jnp.zeros((16,), jnp.float32)
        return 0

    lax.fori_loop(0, ROWS_PT, fill_zeros, 0)

    pltpu.sync_copy(zero_v, acc_sh.at[pl.ds(base, ROWS_PT)])
    pltpu.sync_copy(dst_hbm.at[pl.ds(wid * CPT_DEG, CPT_DEG)], dst_v)
    plsc.subcore_barrier()

    def body(cid, _):
        pltpu.sync_copy(ones_v, acc_sh.at[dst_v.at[cid]], add=True)
        return 0

    lax.fori_loop(0, CPT_DEG, body, 0)
    plsc.subcore_barrier()
    pltpu.sync_copy(acc_sh.at[pl.ds(base, ROWS_PT)],
                    out_hbm.at[c, pl.ds(base, ROWS_PT)])


def _make_agg(width):
    @functools.partial(
        pl.kernel,
        out_type=jax.ShapeDtypeStruct((NC, NPAD, width), jnp.float32),
        mesh=_mesh,
        compiler_params=pltpu.CompilerParams(use_tc_tiling_on_sc=False),
        scratch_types=[
            pltpu.VMEM((CPT_AGG, CH), jnp.int32),
            pltpu.VMEM((CPT_AGG, CH), jnp.int32),
            pltpu.VMEM((2, CH, width), jnp.float32),
            pltpu.VMEM_SHARED((NPAD, width), jnp.float32),
            pltpu.SemaphoreType.DMA((2,)),
        ],
    )
    def agg(xs_hbm, src_hbm, dst_hbm, out_hbm, src_v, dst_v, rows_v, acc_sh,
            sem):
        c = lax.axis_index("c")
        s = lax.axis_index("s")
        base = s * ROWS_PT
        table = xs_hbm.at[c]

        pltpu.sync_copy(table.at[pl.ds(base, ROWS_PT)],
                        acc_sh.at[pl.ds(base, ROWS_PT)])
        pltpu.sync_copy(src_hbm.at[pl.ds(s * CPT_AGG, CPT_AGG)], src_v)
        pltpu.sync_copy(dst_hbm.at[pl.ds(s * CPT_AGG, CPT_AGG)], dst_v)
        plsc.subcore_barrier()

        pltpu.async_copy(table.at[src_v.at[0]], rows_v.at[0], sem.at[0])

        def step(cid, cur, nxt):
            @pl.when(cid + 1 < CPT_AGG)
            def _():
                pltpu.async_copy(table.at[src_v.at[cid + 1]], rows_v.at[nxt],
                                 sem.at[nxt])

            pltpu.make_async_copy(table.at[src_v.at[0]], rows_v.at[cur],
                                  sem.at[cur]).wait()
            pltpu.sync_copy(rows_v.at[cur], acc_sh.at[dst_v.at[cid]],
                            add=True)

        def body(cid, _):
            parity = lax.rem(cid, 2)

            @pl.when(parity == 0)
            def _():
                step(cid, 0, 1)

            @pl.when(parity == 1)
            def _():
                step(cid, 1, 0)

            return 0

        lax.fori_loop(0, CPT_AGG, body, 0)
        plsc.subcore_barrier()
        pltpu.sync_copy(acc_sh.at[pl.ds(base, ROWS_PT)],
                        out_hbm.at[c, pl.ds(base, ROWS_PT)])

    return agg


_agg64 = _make_agg(H1)
_agg32 = _make_agg(H2)


_GPT = (2 * B) // NW
_GC = _GPT // CH


@functools.partial(
    pl.kernel,
    out_type=jax.ShapeDtypeStruct((2 * B, 2 * H2), jnp.float32),
    mesh=_mesh,
    compiler_params=pltpu.CompilerParams(use_tc_tiling_on_sc=False),
    scratch_types=[
        pltpu.VMEM((_GC, CH), jnp.int32),
        pltpu.VMEM((CH, 2 * H2), jnp.float32),
        pltpu.SemaphoreType.DMA,
    ],
)
def _gather_kernel(x2_hbm, idx_hbm, out_hbm, idx_v, rows_v, sem):
    c = lax.axis_index("c")
    s = lax.axis_index("s")
    wid = s * NC + c
    pltpu.sync_copy(idx_hbm.at[wid], idx_v)
    for j in range(_GC):
        pltpu.async_copy(x2_hbm.at[idx_v.at[j]], rows_v, sem).wait()
        pltpu.sync_copy(rows_v, out_hbm.at[pl.ds(wid * _GPT + j * CH, CH)])


def _dinv_from(degp_ref):
    deg = degp_ref[0, :, 0:1] + degp_ref[1, :, 0:1] + 1.0
    return lax.rsqrt(deg)


def _tca_body(xo_ref, xa_ref, w1_ref, degp_ref, out_ref):
    dinv = _dinv_from(degp_ref)
    xw_o = jnp.dot(xo_ref[...], w1_ref[...], preferred_element_type=jnp.float32)
    xw_a = jnp.dot(xa_ref[...], w1_ref[...], preferred_element_type=jnp.float32)
    out_ref[0] = xw_o * dinv
    out_ref[1] = xw_a * dinv


def _tcb_body(p_ref, degp_ref, w2_ref, b1_ref, a1_ref, out_ref):
    dinv = _dinv_from(degp_ref)
    b1 = b1_ref[...]
    a1 = a1_ref[...]
    for half in range(2):
        pre = dinv * p_ref[half] + b1
        h = jnp.where(pre >= 0, pre, a1 * pre)
        out_ref[half] = jnp.dot(h, w2_ref[...],
                                preferred_element_type=jnp.float32) * dinv


def _tcc_body(p_ref, degp_ref, b2_ref, a2_ref, wm0_ref, bm0_ref,
              wm1_ref, bm1_ref, wb_ref, bb_ref, x2_ref, small_ref):
    dinv = _dinv_from(degp_ref)
    b2 = b2_ref[...]
    a2 = a2_ref[...]
    pre_o = dinv * p_ref[0] + b2
    pre_a = dinv * p_ref[1] + b2
    x2 = jnp.concatenate([jnp.where(pre_o >= 0, pre_o, a2 * pre_o),
                          jnp.where(pre_a >= 0, pre_a, a2 * pre_a)], axis=1)
    x2_ref[...] = x2

    mask = lax.broadcasted_iota(jnp.int32, (NPAD, 1), 0) < N
    x2m = jnp.where(mask, x2, 0.0)
    ones_n = jnp.full((NPAD, 1), 1.0, jnp.float32)
    sums_col = lax.dot_general(x2m, ones_n, (((0,), (0,)), ((), ())),
                               preferred_element_type=jnp.float32)
    inv_n = 1.0 / jnp.float32(N)
    sig_o = jax.nn.sigmoid(sums_col[:H2] * inv_n)
    sig_a = jax.nn.sigmoid(sums_col[H2:] * inv_n)
    wm1 = wm1_ref[...]
    bm1_col = _row_to_col(bm1_ref)
    h_os = lax.dot_general(wm1, sig_o, (((0,), (0,)), ((), ())),
                           preferred_element_type=jnp.float32) + bm1_col
    h_os_a = lax.dot_general(wm1, sig_a, (((0,), (0,)), ((), ())),
                             preferred_element_type=jnp.float32) + bm1_col
    wb = wb_ref[...]
    q_os = jnp.dot(wb, h_os, preferred_element_type=jnp.float32)
    q_osa = jnp.dot(wb, h_os_a, preferred_element_type=jnp.float32)
    ones_h = jnp.full((H2, 1), 1.0, jnp.float32)
    wm0s = jnp.dot(wm0_ref[...], ones_h, preferred_element_type=jnp.float32)
    z = jnp.zeros((H2, 1), jnp.float32)
    cols = jnp.concatenate([
        jnp.concatenate([wm0s, z], axis=0),
        jnp.concatenate([z, wm0s], axis=0),
        jnp.concatenate([q_os, z], axis=0),
        jnp.concatenate([z, q_os], axis=0),
        jnp.concatenate([z, q_osa], axis=0),
        jnp.concatenate([q_osa, z], axis=0),
        jnp.zeros((2 * H2, 2), jnp.float32),
    ], axis=1)
    smalls = jnp.dot(x2, cols, preferred_element_type=jnp.float32)
    bm0s = jnp.sum(bm0_ref[...])
    bbv = bb_ref[0, 0]
    brow = jnp.concatenate([
        jnp.full((1, 2), 1.0, jnp.float32) * bm0s,
        jnp.full((1, 4), 1.0, jnp.float32) * bbv,
        jnp.zeros((1, 2), jnp.float32),
    ], axis=1)
    small_ref[...] = smalls + brow


def _row_to_col(b_ref):
    return lax.dot_general(b_ref[...], jnp.full((1, 1), 1.0, jnp.float32),
                           (((0,), (0,)), ((), ())),
                           preferred_element_type=jnp.float32)


def _tcd_body(e_ref, wd1_ref, bd1_ref, wd2_ref, bd2_ref, log1_ref, log_ref):
    e1 = e_ref[:B, :H2]
    e2 = e_ref[B:, :H2]
    feature = jnp.concatenate([e1 + e2, e1 * e2, e1, e2], axis=1)
    log1 = jnp.dot(feature, wd1_ref[...], preferred_element_type=jnp.float32)
    log1 = jnp.maximum(log1 + bd1_ref[...], 0.0)
    log1_ref[...] = log1
    log_ref[...] = jnp.dot(log1, wd2_ref[...],
                           preferred_element_type=jnp.float32) + bd2_ref[...]


def _f32(shape):
    return jax.ShapeDtypeStruct(shape, jnp.float32)


def kernel(x_o, x_a, W1, b1, a1, W2, b2, a2, Wm0, bm0, Wm1, bm1, Wd1, bd1,
           Wd2, bd2, Wb, bb, edge_index, idx):
    f32 = jnp.float32
    xo_p = jnp.zeros((NPAD, F_IN), f32).at[:N].set(x_o)
    xa_p = jnp.zeros((NPAD, F_IN), f32).at[:N].set(x_a)
    ei_pad = jnp.full((2, EPAD - E), DUMMY, jnp.int32)
    ei = jnp.concatenate([edge_index.astype(jnp.int32), ei_pad], axis=1)
    src2 = ei[0].reshape(NCHUNKS, CH)
    dst2 = ei[1].reshape(NCHUNKS, CH)

    degp = _deg_kernel(dst2)
    xs1 = pl.pallas_call(_tca_body, out_shape=_f32((NC, NPAD, H1)))(
        xo_p, xa_p, W1, degp)
    p1 = _agg64(xs1, src2, dst2)
    xs2 = pl.pallas_call(_tcb_body, out_shape=_f32((NC, NPAD, H2)))(
        p1, degp, W2, b1.reshape(1, H1), a1.reshape(1, H1))
    p2 = _agg32(xs2, src2, dst2)
    x2, smalls = pl.pallas_call(
        _tcc_body, out_shape=(_f32((NPAD, 2 * H2)), _f32((NPAD, 8))))(
            p2, degp, b2.reshape(1, H2), a2.reshape(1, H2), Wm0,
            bm0.reshape(1, H2), Wm1, bm1.reshape(1, H2), Wb[0],
            bb.reshape(1, 1))

    gidx = jnp.concatenate([idx[0], idx[1] + 240]).astype(jnp.int32)
    gidx3 = gidx.reshape(NW, _GC, CH)
    e_rows = _gather_kernel(x2, gidx3)
    log1, log = pl.pallas_call(
        _tcd_body, out_shape=(_f32((B, H1)), _f32((B, 1))))(
            e_rows, Wd1, bd1.reshape(1, H1), Wd2, bd2.reshape(1, 1))

    sc = smalls[:N]
    logits = jnp.concatenate([sc[:, 0], sc[:, 1]])[None, :]
    return (log, sc[:, 2:4], sc[:, 4:6], x2[:N, :H2], logits, log1)
